# baseline (speedup 1.0000x reference)
import jax
import jax.numpy as jnp
from jax.experimental import pallas as pl
from jax.experimental.pallas import tpu as pltpu

N, NFEAT, NHID, NCODE, NCLASS = 10000, 128, 64, 32, 16
BM = 400


def _dot(a, b):
    return jax.lax.dot(a, b, preferred_element_type=jnp.float32)


def _fused(x_ref, adj_ref, Wgc1_ref, bgc1_ref, W11_ref, b11_ref, W12_ref,
           b12_ref, W2_ref, b2_ref, Wgc2_ref, bgc2_ref, eps_ref,
           out_ref, rhs_ref, u_ref):
    p, m = pl.program_id(0), pl.program_id(1)

    @pl.when((p == 0) & (m == 0))
    def _():
        rhs_ref[...] = _dot(x_ref[...], Wgc1_ref[...])

    @pl.when((p == 1) & (m == 0))
    def _():
        rhs_ref[...] = jnp.concatenate(
            [u_ref[...], jnp.zeros((N, NHID - NCLASS), jnp.float32)], axis=1)

    acc = _dot(adj_ref[...], rhs_ref[...])

    @pl.when(p == 0)
    def _():
        h1 = jnp.maximum(acc + bgc1_ref[...], 0.0)
        mu = _dot(h1, W11_ref[...]) + b11_ref[...]
        logvar = _dot(h1, W12_ref[...]) + b12_ref[...]
        z = mu + eps_ref[...] * jnp.exp(logvar)
        x1a = jnp.maximum(_dot(z, W2_ref[...]) + b2_ref[...], 0.0)
        u_ref[pl.ds(m * BM, BM), :] = (_dot(x1a, Wgc2_ref[0:NHID, :])
                                       + _dot(h1, Wgc2_ref[NHID:, :]))

    @pl.when(p == 1)
    def _():
        o = acc[:, 0:NCLASS] + bgc2_ref[...]
        mx = jnp.max(o, axis=1, keepdims=True)
        s = o - mx
        out_ref[...] = s - jnp.log(jnp.sum(jnp.exp(s), axis=1, keepdims=True))


def kernel(x, adj, W_gc1, b_gc1, W11, b11, W12, b12, W2, b2, W_gc2, b_gc2, eps):
    full = lambda shape: pl.BlockSpec(shape, lambda p, m: (0, 0))

    out = pl.pallas_call(
        _fused,
        grid=(2, N // BM),
        in_specs=[
            full((N, NFEAT)),
            pl.BlockSpec((BM, N), lambda p, m: (m, 0)),
            full((NFEAT, NHID)),
            full((1, NHID)),
            full((NHID, NCODE)),
            full((1, NCODE)),
            full((NHID, NCODE)),
            full((1, NCODE)),
            full((NCODE, NHID)),
            full((1, NHID)),
            full((2 * NHID, NCLASS)),
            full((1, NCLASS)),
            pl.BlockSpec((BM, NCODE), lambda p, m: ((1 - p) * m, 0)),
        ],
        out_specs=pl.BlockSpec((BM, NCLASS), lambda p, m: (p * m, 0)),
        out_shape=jax.ShapeDtypeStruct((N, NCLASS), jnp.float32),
        scratch_shapes=[pltpu.VMEM((N, NHID), jnp.float32),
                        pltpu.VMEM((N, NCLASS), jnp.float32)],
        compiler_params=pltpu.CompilerParams(
            dimension_semantics=("arbitrary", "parallel"),
            vmem_limit_bytes=64 * 1024 * 1024),
    )(x, adj, W_gc1, b_gc1.reshape(1, -1), W11, b11.reshape(1, -1),
      W12, b12.reshape(1, -1), W2, b2.reshape(1, -1), W_gc2,
      b_gc2.reshape(1, -1), eps)
    return out


# speedup vs baseline: 1.0053x; 1.0053x over previous
import jax
import jax.numpy as jnp
from jax.experimental import pallas as pl
from jax.experimental.pallas import tpu as pltpu

N, NFEAT, NHID, NCODE, NCLASS = 10000, 128, 64, 32, 16
BM = 400


def _dot(a, b):
    return jax.lax.dot(a, b, preferred_element_type=jnp.float32)


def _fused(x_ref, adj_ref, Wgc1_ref, bgc1_ref, W11_ref, b11_ref, W12_ref,
           b12_ref, W2_ref, b2_ref, Wgc2_ref, bgc2_ref, eps_ref,
           out_ref, rhs_ref):
    p, m = pl.program_id(0), pl.program_id(1)

    @pl.when((p == 0) & (m == 0))
    def _():
        rhs_ref[0] = _dot(x_ref[...], Wgc1_ref[...])

    acc = _dot(adj_ref[...], rhs_ref[p])

    @pl.when(p == 0)
    def _():
        h1 = jnp.maximum(acc + bgc1_ref[...], 0.0)
        mu = _dot(h1, W11_ref[...]) + b11_ref[...]
        logvar = _dot(h1, W12_ref[...]) + b12_ref[...]
        z = mu + eps_ref[...] * jnp.exp(logvar)
        x1a = jnp.maximum(_dot(z, W2_ref[...]) + b2_ref[...], 0.0)
        u = _dot(x1a, Wgc2_ref[0:NHID, :]) + _dot(h1, Wgc2_ref[NHID:, :])
        rhs_ref[1, pl.ds(m * BM, BM), :] = jnp.concatenate(
            [u, jnp.zeros((BM, NHID - NCLASS), jnp.float32)], axis=1)

    @pl.when(p == 1)
    def _():
        o = acc[:, 0:NCLASS] + bgc2_ref[...]
        mx = jnp.max(o, axis=1, keepdims=True)
        s = o - mx
        out_ref[...] = s - jnp.log(jnp.sum(jnp.exp(s), axis=1, keepdims=True))


def kernel(x, adj, W_gc1, b_gc1, W11, b11, W12, b12, W2, b2, W_gc2, b_gc2, eps):
    full = lambda shape: pl.BlockSpec(shape, lambda p, m: (0, 0))

    out = pl.pallas_call(
        _fused,
        grid=(2, N // BM),
        in_specs=[
            full((N, NFEAT)),
            pl.BlockSpec((BM, N), lambda p, m: (m, 0)),
            full((NFEAT, NHID)),
            full((1, NHID)),
            full((NHID, NCODE)),
            full((1, NCODE)),
            full((NHID, NCODE)),
            full((1, NCODE)),
            full((NCODE, NHID)),
            full((1, NHID)),
            full((2 * NHID, NCLASS)),
            full((1, NCLASS)),
            pl.BlockSpec((BM, NCODE), lambda p, m: ((1 - p) * m, 0)),
        ],
        out_specs=pl.BlockSpec((BM, NCLASS), lambda p, m: (p * m, 0)),
        out_shape=jax.ShapeDtypeStruct((N, NCLASS), jnp.float32),
        scratch_shapes=[pltpu.VMEM((2, N, NHID), jnp.float32)],
        compiler_params=pltpu.CompilerParams(
            dimension_semantics=("arbitrary", "arbitrary"),
            vmem_limit_bytes=64 * 1024 * 1024),
    )(x, adj, W_gc1, b_gc1.reshape(1, -1), W11, b11.reshape(1, -1),
      W12, b12.reshape(1, -1), W2, b2.reshape(1, -1), W_gc2,
      b_gc2.reshape(1, -1), eps)
    return out


# speedup vs baseline: 1.0117x; 1.0063x over previous
import jax
import jax.numpy as jnp
from jax.experimental import pallas as pl
from jax.experimental.pallas import tpu as pltpu

N, NFEAT, NHID, NCODE, NCLASS = 10000, 128, 64, 32, 16
BM = 400


def _dot(a, b):
    return jax.lax.dot(a, b, preferred_element_type=jnp.float32)


def _fused(x_ref, adj_ref, Wgc1_ref, bgc1_ref, W11_ref, b11_ref, W12_ref,
           b12_ref, W2_ref, b2_ref, Wgc2_ref, bgc2_ref, eps_ref,
           out_ref, rhs_ref, u_ref):
    p, m = pl.program_id(0), pl.program_id(1)

    @pl.when((p == 0) & (m == 0))
    def _():
        rhs_ref[...] = _dot(x_ref[...], Wgc1_ref[...])

    @pl.when((p == 1) & (m == 0))
    def _():
        rhs_ref[...] = jnp.concatenate(
            [u_ref[...], jnp.zeros((N, NHID - NCLASS), jnp.float32)], axis=1)

    acc = _dot(adj_ref[...], rhs_ref[...])

    @pl.when(p == 0)
    def _():
        h1 = jnp.maximum(acc + bgc1_ref[...], 0.0)
        mu = _dot(h1, W11_ref[...]) + b11_ref[...]
        logvar = _dot(h1, W12_ref[...]) + b12_ref[...]
        z = mu + eps_ref[...] * jnp.exp(logvar)
        x1a = jnp.maximum(_dot(z, W2_ref[...]) + b2_ref[...], 0.0)
        u_ref[pl.ds(m * BM, BM), :] = (_dot(x1a, Wgc2_ref[0:NHID, :])
                                       + _dot(h1, Wgc2_ref[NHID:, :]))

    @pl.when(p == 1)
    def _():
        o = acc[:, 0:NCLASS] + bgc2_ref[...]
        mx = jnp.max(o, axis=1, keepdims=True)
        s = o - mx
        out_ref[...] = s - jnp.log(jnp.sum(jnp.exp(s), axis=1, keepdims=True))


def kernel(x, adj, W_gc1, b_gc1, W11, b11, W12, b12, W2, b2, W_gc2, b_gc2, eps):
    full = lambda shape: pl.BlockSpec(shape, lambda p, m: (0, 0))

    out = pl.pallas_call(
        _fused,
        grid=(2, N // BM),
        in_specs=[
            full((N, NFEAT)),
            pl.BlockSpec((BM, N), lambda p, m: (m, 0)),
            full((NFEAT, NHID)),
            full((1, NHID)),
            full((NHID, NCODE)),
            full((1, NCODE)),
            full((NHID, NCODE)),
            full((1, NCODE)),
            full((NCODE, NHID)),
            full((1, NHID)),
            full((2 * NHID, NCLASS)),
            full((1, NCLASS)),
            pl.BlockSpec((BM, NCODE), lambda p, m: ((1 - p) * m, 0)),
        ],
        out_specs=pl.BlockSpec((BM, NCLASS), lambda p, m: (p * m, 0)),
        out_shape=jax.ShapeDtypeStruct((N, NCLASS), jnp.float32),
        scratch_shapes=[pltpu.VMEM((N, NHID), jnp.float32),
                        pltpu.VMEM((N, NCLASS), jnp.float32)],
        compiler_params=pltpu.CompilerParams(
            dimension_semantics=("arbitrary", "arbitrary"),
            vmem_limit_bytes=64 * 1024 * 1024),
    )(x, adj, W_gc1, b_gc1.reshape(1, -1), W11, b11.reshape(1, -1),
      W12, b12.reshape(1, -1), W2, b2.reshape(1, -1), W_gc2,
      b_gc2.reshape(1, -1), eps)
    return out
